# Initial kernel scaffold; baseline (speedup 1.0000x reference)
import jax
import jax.numpy as jnp
from jax.experimental import pallas as pl


def kernel(words, table, W1, b1, W2, b2, Wout, bout):
    raise NotImplementedError("write your pallas kernel here")


# speedup vs baseline: 2.4475x; 2.4475x over previous
import functools

import jax
import jax.numpy as jnp
from jax import lax
from jax.experimental import pallas as pl
from jax.experimental.pallas import tpu as pltpu
from jax.experimental.pallas import tpu_sc as plsc

_NWORDS = 1000000
_EMB = 64
_NIDX = 16384 * 50

_NC, _NS, _NL = 2, 16, 16
_NW = _NC * _NS
_BPW = _NIDX // _NW

_RB = 8192
_NBLK = (_NWORDS + _RB - 1) // _RB


def _rowsum_body(x_ref, o_ref):
    o_ref[...] = jnp.sum(x_ref[...], axis=1)


_rowsum_call = pl.pallas_call(
    _rowsum_body,
    grid=(_NBLK,),
    in_specs=[pl.BlockSpec((_RB, _EMB), lambda i: (i, 0))],
    out_specs=pl.BlockSpec((_RB,), lambda i: (i,)),
    out_shape=jax.ShapeDtypeStruct((_NWORDS,), jnp.float32),
)


@functools.partial(
    pl.kernel,
    mesh=plsc.VectorSubcoreMesh(core_axis_name="c", subcore_axis_name="s"),
    out_type=jax.ShapeDtypeStruct((_NW, _NL), jnp.float32),
    scratch_types=[
        pltpu.VMEM((_BPW,), jnp.int32),
        pltpu.VMEM((_BPW,), jnp.float32),
        pltpu.VMEM((_NL,), jnp.float32),
        pltpu.SemaphoreType.DMA,
    ],
)
def _sc_gather_sum(words_hbm, rowsums_hbm, out_hbm, idx_v, vals_v, acc_v, sem):
    wid = lax.axis_index("s") * _NC + lax.axis_index("c")
    base = wid * _BPW
    pltpu.sync_copy(words_hbm.at[pl.ds(base, _BPW)], idx_v)
    pltpu.async_copy(rowsums_hbm.at[idx_v], vals_v, sem).wait()

    def body(i, acc):
        return acc + vals_v[pl.ds(i * _NL, _NL)]

    acc = lax.fori_loop(0, _BPW // _NL, body, jnp.zeros((_NL,), jnp.float32))
    acc_v[...] = acc
    pltpu.sync_copy(acc_v, out_hbm.at[wid])


def _mlp_body(p_ref, w1_ref, b1_ref, w2_ref, b2_ref, wo_ref, bo_ref, o_ref):
    s = jnp.sum(p_ref[...])
    h1 = jnp.tanh(s * w1_ref[...] + b1_ref[...])
    h2 = jnp.tanh(
        jnp.dot(h1, w2_ref[...], preferred_element_type=jnp.float32) + b2_ref[...]
    )
    o_ref[...] = (
        jnp.dot(h2, wo_ref[...], preferred_element_type=jnp.float32) + bo_ref[...]
    )


def _mlp_call(partials, W1, b1, W2, b2, Wout, bout):
    return pl.pallas_call(
        _mlp_body,
        out_shape=jax.ShapeDtypeStruct((1, bout.shape[-1]), jnp.float32),
    )(partials, W1, b1, W2, b2, Wout, bout)


def kernel(words, table, W1, b1, W2, b2, Wout, bout):
    words_flat = words.reshape(-1).astype(jnp.int32)
    rowsums = _rowsum_call(table)
    partials = _sc_gather_sum(words_flat, rowsums)
    return _mlp_call(
        partials,
        W1,
        b1.reshape(1, -1),
        W2,
        b2.reshape(1, -1),
        Wout,
        bout.reshape(1, -1),
    )
